# baseline (speedup 1.0000x reference)
import functools

import jax
import jax.numpy as jnp
from jax import lax
from jax.experimental import pallas as pl
from jax.experimental.pallas import tpu as pltpu
from jax.experimental.pallas import tpu_sc as plsc

B = 16384
IN = 2048
NCLS = 3
H = 341
HP = 384
OUTP = 128
T = 256
TSHIFT = 8
NP = B + 4 * T
NT = NP // T
NTP = 80
NCORE = 2
NSUB = 16
NW = NCORE * NSUB
CHUNK = B // NW
VPC = CHUNK // 16
GR = 16

def _route_dispatch_body(cls_hbm, emb_hbm, inv_hbm, tcls_hbm, xs_hbm,
                         cls_v, pos_v, xbuf_v, tcls_v, sem):
    wid = lax.axis_index("s") * NCORE + lax.axis_index("c")
    base = wid * CHUNK
    pltpu.sync_copy(cls_hbm, cls_v)

    zero = jnp.int32(0)
    t0 = wid * VPC

    def count_body(i, carry):
        c0, c1 = carry
        v = cls_v[pl.ds(i * 16, 16)]
        return (c0 + jnp.sum((v == 0).astype(jnp.int32)),
                c1 + jnp.sum((v == 1).astype(jnp.int32)))

    pre0, pre1 = lax.fori_loop(0, t0, count_body, (zero, zero))
    rest0, rest1 = lax.fori_loop(t0, B // 16, count_body, (zero, zero))
    tot0 = pre0 + rest0
    tot1 = pre1 + rest1
    pre2 = t0 * 16 - pre0 - pre1

    cap0 = ((tot0 + (T - 1)) >> TSHIFT) << TSHIFT
    cap1 = ((tot1 + (T - 1)) >> TSHIFT) << TSHIFT
    o1 = cap0
    o2 = cap0 + cap1

    @pl.when(wid == 0)
    def _emit_tile_classes():
        for k in range(NTP // 16):
            tv = (lax.iota(jnp.int32, 16) + 16 * k) * T
            tcls_v[pl.ds(16 * k, 16)] = (
                (tv >= o1).astype(jnp.int32) + (tv >= o2).astype(jnp.int32))
        pltpu.sync_copy(tcls_v, tcls_hbm)

    def pos_body(g, carry):
        b0, b1, b2 = carry
        v = cls_v[pl.ds((t0 + g) * 16, 16)]
        m0 = v == 0
        m1 = v == 1
        m2 = v == 2
        r0 = jnp.cumsum(m0.astype(jnp.int32))
        r1 = jnp.cumsum(m1.astype(jnp.int32))
        r2 = jnp.cumsum(m2.astype(jnp.int32))
        pos = jnp.where(m0, b0 + r0, jnp.where(m1, b1 + r1, b2 + r2)) - 1
        pos_v[g] = pos
        return (b0 + r0[15], b1 + r1[15], b2 + r2[15])

    lax.fori_loop(0, VPC, pos_body, (pre0, o1 + pre1, o2 + pre2))
    pltpu.sync_copy(pos_v, inv_hbm.at[wid])

    def dma_body(g, _):
        pltpu.sync_copy(emb_hbm.at[pl.ds(base + g * GR, GR)], xbuf_v)
        pltpu.async_copy(xbuf_v, xs_hbm.at[pos_v.at[g]], sem).wait()
        return 0

    lax.fori_loop(0, VPC, dma_body, 0)


def _mlp_body(tcls_ref, x_ref, w1_ref, b1_ref, w2_ref, b2_ref,
              w3_ref, b3_ref, w4_ref, b4_ref, o_ref):
    x = x_ref[...]
    h = jnp.dot(x, w1_ref[0], preferred_element_type=jnp.float32) + b1_ref[0]
    h = jnp.maximum(h, 0.0)
    h = jnp.dot(h, w2_ref[0], preferred_element_type=jnp.float32) + b2_ref[0]
    h = jnp.maximum(h, 0.0)
    h = jnp.dot(h, w3_ref[0], preferred_element_type=jnp.float32) + b3_ref[0]
    h = jnp.maximum(h, 0.0)
    o_ref[...] = (jnp.dot(h, w4_ref[0], preferred_element_type=jnp.float32)
                  + b4_ref[0])


_mlp_grid = pltpu.PrefetchScalarGridSpec(
    num_scalar_prefetch=1,
    grid=(NT,),
    in_specs=[
        pl.BlockSpec((T, IN), lambda t, tc: (t, 0)),
        pl.BlockSpec((1, IN, HP), lambda t, tc: (tc[t], 0, 0)),
        pl.BlockSpec((1, 1, HP), lambda t, tc: (tc[t], 0, 0)),
        pl.BlockSpec((1, HP, HP), lambda t, tc: (tc[t], 0, 0)),
        pl.BlockSpec((1, 1, HP), lambda t, tc: (tc[t], 0, 0)),
        pl.BlockSpec((1, HP, HP), lambda t, tc: (tc[t], 0, 0)),
        pl.BlockSpec((1, 1, HP), lambda t, tc: (tc[t], 0, 0)),
        pl.BlockSpec((1, HP, OUTP), lambda t, tc: (tc[t], 0, 0)),
        pl.BlockSpec((1, 1, OUTP), lambda t, tc: (tc[t], 0, 0)),
    ],
    out_specs=pl.BlockSpec((T, OUTP), lambda t, tc: (t, 0)),
)

_mlp_call = pl.pallas_call(
    _mlp_body,
    grid_spec=_mlp_grid,
    out_shape=jax.ShapeDtypeStruct((NP, OUTP), jnp.float32),
    compiler_params=pltpu.CompilerParams(
        dimension_semantics=("arbitrary",)),
)


def _unsort_body(res_hbm, inv_hbm, out_hbm, res_v, inv_v, out_v):
    wid = lax.axis_index("s") * NCORE + lax.axis_index("c")
    pltpu.sync_copy(res_hbm, res_v)
    pltpu.sync_copy(inv_hbm.at[wid], inv_v)

    def body(g, _):
        out_v[g] = plsc.load_gather(res_v, [inv_v[g]])
        return 0

    lax.fori_loop(0, VPC, body, 0)
    pltpu.sync_copy(out_v, out_hbm.at[wid])


@functools.lru_cache(maxsize=1)
def _sc_kernels():
    mesh = plsc.VectorSubcoreMesh(
        core_axis_name="c", subcore_axis_name="s",
        num_cores=NCORE, num_subcores=NSUB)
    sc_params = pltpu.CompilerParams(needs_layout_passes=False)
    route = pl.kernel(
        _route_dispatch_body,
        compiler_params=sc_params,
        out_type=(
            jax.ShapeDtypeStruct((NW, VPC, 16), jnp.int32),
            jax.ShapeDtypeStruct((NTP,), jnp.int32),
            jax.ShapeDtypeStruct((NP, IN), jnp.float32),
        ),
        mesh=mesh,
        scratch_types=(
            pltpu.VMEM((B,), jnp.int32),
            pltpu.VMEM((VPC, 16), jnp.int32),
            pltpu.VMEM((GR, IN), jnp.float32),
            pltpu.VMEM((NTP,), jnp.int32),
            pltpu.SemaphoreType.DMA,
        ),
    )
    unsort = pl.kernel(
        _unsort_body,
        compiler_params=sc_params,
        out_type=jax.ShapeDtypeStruct((NW, VPC, 16), jnp.float32),
        mesh=mesh,
        scratch_types=(
            pltpu.VMEM((NP,), jnp.float32),
            pltpu.VMEM((VPC, 16), jnp.int32),
            pltpu.VMEM((VPC, 16), jnp.float32),
        ),
    )
    return route, unsort


def _pad_params(params):
    w1 = jnp.stack([jnp.pad(params[c][0][0], ((0, 0), (0, HP - H)))
                    for c in range(NCLS)])
    b1 = jnp.stack([jnp.pad(params[c][0][1], (0, HP - H))[None]
                    for c in range(NCLS)])
    w2 = jnp.stack([jnp.pad(params[c][1][0], ((0, HP - H), (0, HP - H)))
                    for c in range(NCLS)])
    b2 = jnp.stack([jnp.pad(params[c][1][1], (0, HP - H))[None]
                    for c in range(NCLS)])
    w3 = jnp.stack([jnp.pad(params[c][2][0], ((0, HP - H), (0, HP - H)))
                    for c in range(NCLS)])
    b3 = jnp.stack([jnp.pad(params[c][2][1], (0, HP - H))[None]
                    for c in range(NCLS)])
    w4 = jnp.stack([jnp.pad(params[c][3][0], ((0, HP - H), (0, OUTP - 1)))
                    for c in range(NCLS)])
    b4 = jnp.stack([jnp.pad(params[c][3][1], (0, OUTP - 1))[None]
                    for c in range(NCLS)])
    return w1, b1, w2, b2, w3, b3, w4, b4


def kernel(graph_emb, class_idx, params):
    w1, b1, w2, b2, w3, b3, w4, b4 = _pad_params(params)
    route, unsort = _sc_kernels()
    inv, tcls, xs = route(class_idx, graph_emb)
    res = _mlp_call(tcls[:NT], xs, w1, b1, w2, b2, w3, b3, w4, b4)
    out = unsort(res[:, 0], inv)
    return out.reshape(B)


# speedup vs baseline: 1.0669x; 1.0669x over previous
import functools

import jax
import jax.numpy as jnp
from jax import lax
from jax.experimental import pallas as pl
from jax.experimental.pallas import tpu as pltpu
from jax.experimental.pallas import tpu_sc as plsc

B = 16384
IN = 2048
NCLS = 3
H = 341
HP = 384
OUTP = 128
T = 256
TSHIFT = 8
NP = B + 4 * T
NT = NP // T
NTP = 80
NCORE = 2
NSUB = 16
NW = NCORE * NSUB
CHUNK = B // NW
VPC = CHUNK // 16
GR = 16

def _route_dispatch_body(cls_hbm, emb_hbm, inv_hbm, tcls_hbm, xs_hbm,
                         cls_v, pos_v, xa_v, xb_v, xc_v, tcls_v,
                         sin0, sin1, sin2, sout0, sout1, sout2):
    wid = lax.axis_index("s") * NCORE + lax.axis_index("c")
    base = wid * CHUNK
    pltpu.sync_copy(cls_hbm, cls_v)

    vzero = jnp.zeros((16,), jnp.int32)
    t0 = wid * VPC

    def count_body(i, carry):
        a0, a1 = carry
        v = cls_v[pl.ds(i * 16, 16)]
        return (a0 + (v == 0).astype(jnp.int32),
                a1 + (v == 1).astype(jnp.int32))

    acc0, acc1 = lax.fori_loop(0, t0, count_body, (vzero, vzero))
    pre0 = jnp.sum(acc0)
    pre1 = jnp.sum(acc1)
    acc0, acc1 = lax.fori_loop(t0, B // 16, count_body, (acc0, acc1))
    tot0 = jnp.sum(acc0)
    tot1 = jnp.sum(acc1)
    pre2 = t0 * 16 - pre0 - pre1

    cap0 = ((tot0 + (T - 1)) >> TSHIFT) << TSHIFT
    cap1 = ((tot1 + (T - 1)) >> TSHIFT) << TSHIFT
    o1 = cap0
    o2 = cap0 + cap1

    @pl.when(wid == 0)
    def _emit_tile_classes():
        for k in range(NTP // 16):
            tv = (lax.iota(jnp.int32, 16) + 16 * k) * T
            tcls_v[pl.ds(16 * k, 16)] = (
                (tv >= o1).astype(jnp.int32) + (tv >= o2).astype(jnp.int32))
        pltpu.sync_copy(tcls_v, tcls_hbm)

    def pos_body(g, carry):
        b0, b1, b2 = carry
        v = cls_v[pl.ds((t0 + g) * 16, 16)]
        m0 = v == 0
        m1 = v == 1
        m2 = v == 2
        r0 = jnp.cumsum(m0.astype(jnp.int32))
        r1 = jnp.cumsum(m1.astype(jnp.int32))
        r2 = jnp.cumsum(m2.astype(jnp.int32))
        pos = jnp.where(m0, b0 + r0, jnp.where(m1, b1 + r1, b2 + r2)) - 1
        pos_v[g] = pos
        return (b0 + r0[15], b1 + r1[15], b2 + r2[15])

    lax.fori_loop(0, VPC, pos_body, (pre0, o1 + pre1, o2 + pre2))
    pltpu.sync_copy(pos_v, inv_hbm.at[wid])

    bufs = (xa_v, xb_v, xc_v)
    sin = (sin0, sin1, sin2)
    sout = (sout0, sout1, sout2)

    def start_in(g, b):
        pltpu.async_copy(emb_hbm.at[pl.ds(base + g * GR, GR)], bufs[b], sin[b])

    def wait_in(g, b):
        pltpu.make_async_copy(
            emb_hbm.at[pl.ds(base + g * GR, GR)], bufs[b], sin[b]).wait()

    def start_out(g, b):
        pltpu.async_copy(bufs[b], xs_hbm.at[pos_v.at[g]], sout[b])

    def wait_out(g, b):
        pltpu.make_async_copy(
            bufs[b], xs_hbm.at[pos_v.at[g]], sout[b]).wait()

    start_in(0, 0)

    def ring_body(jj, _):
        for b in range(3):
            g = 3 * jj + b
            if b == 2:
                wait_out(g - 2, 0)
            else:
                @pl.when(jj >= 1)
                def _():
                    wait_out(g - 2, b + 1)
            start_in(g + 1, (b + 1) % 3)
            wait_in(g, b)
            start_out(g, b)
        return 0

    lax.fori_loop(0, 10, ring_body, 0)
    wait_out(28, 1)
    start_in(31, 1)
    wait_in(30, 0)
    start_out(30, 0)
    wait_out(29, 2)
    wait_in(31, 1)
    start_out(31, 1)
    wait_out(30, 0)
    wait_out(31, 1)


def _mlp_body(tcls_ref, x_ref, w1_ref, b1_ref, w2_ref, b2_ref,
              w3_ref, b3_ref, w4_ref, b4_ref, o_ref):
    x = x_ref[...]
    h = jnp.dot(x, w1_ref[0], preferred_element_type=jnp.float32) + b1_ref[0]
    h = jnp.maximum(h, 0.0)
    h = jnp.dot(h, w2_ref[0], preferred_element_type=jnp.float32) + b2_ref[0]
    h = jnp.maximum(h, 0.0)
    h = jnp.dot(h, w3_ref[0], preferred_element_type=jnp.float32) + b3_ref[0]
    h = jnp.maximum(h, 0.0)
    o_ref[...] = (jnp.dot(h, w4_ref[0], preferred_element_type=jnp.float32)
                  + b4_ref[0])


_mlp_grid = pltpu.PrefetchScalarGridSpec(
    num_scalar_prefetch=1,
    grid=(NT,),
    in_specs=[
        pl.BlockSpec((T, IN), lambda t, tc: (t, 0)),
        pl.BlockSpec((1, IN, HP), lambda t, tc: (tc[t], 0, 0)),
        pl.BlockSpec((1, 1, HP), lambda t, tc: (tc[t], 0, 0)),
        pl.BlockSpec((1, HP, HP), lambda t, tc: (tc[t], 0, 0)),
        pl.BlockSpec((1, 1, HP), lambda t, tc: (tc[t], 0, 0)),
        pl.BlockSpec((1, HP, HP), lambda t, tc: (tc[t], 0, 0)),
        pl.BlockSpec((1, 1, HP), lambda t, tc: (tc[t], 0, 0)),
        pl.BlockSpec((1, HP, OUTP), lambda t, tc: (tc[t], 0, 0)),
        pl.BlockSpec((1, 1, OUTP), lambda t, tc: (tc[t], 0, 0)),
    ],
    out_specs=pl.BlockSpec((T, OUTP), lambda t, tc: (t, 0)),
)

_mlp_call = pl.pallas_call(
    _mlp_body,
    grid_spec=_mlp_grid,
    out_shape=jax.ShapeDtypeStruct((NP, OUTP), jnp.float32),
    compiler_params=pltpu.CompilerParams(
        dimension_semantics=("arbitrary",)),
)


def _unsort_body(res_hbm, inv_hbm, out_hbm, res_v, inv_v, out_v):
    wid = lax.axis_index("s") * NCORE + lax.axis_index("c")
    pltpu.sync_copy(res_hbm, res_v)
    pltpu.sync_copy(inv_hbm.at[wid], inv_v)

    def body(g, _):
        out_v[g] = plsc.load_gather(res_v, [inv_v[g]])
        return 0

    lax.fori_loop(0, VPC, body, 0)
    pltpu.sync_copy(out_v, out_hbm.at[wid])


@functools.lru_cache(maxsize=1)
def _sc_kernels():
    mesh = plsc.VectorSubcoreMesh(
        core_axis_name="c", subcore_axis_name="s",
        num_cores=NCORE, num_subcores=NSUB)
    sc_params = pltpu.CompilerParams(needs_layout_passes=False)
    route = pl.kernel(
        _route_dispatch_body,
        compiler_params=sc_params,
        out_type=(
            jax.ShapeDtypeStruct((NW, VPC, 16), jnp.int32),
            jax.ShapeDtypeStruct((NTP,), jnp.int32),
            jax.ShapeDtypeStruct((NP, IN), jnp.float32),
        ),
        mesh=mesh,
        scratch_types=(
            pltpu.VMEM((B,), jnp.int32),
            pltpu.VMEM((VPC, 16), jnp.int32),
            pltpu.VMEM((GR, IN), jnp.float32),
            pltpu.VMEM((GR, IN), jnp.float32),
            pltpu.VMEM((GR, IN), jnp.float32),
            pltpu.VMEM((NTP,), jnp.int32),
            pltpu.SemaphoreType.DMA,
            pltpu.SemaphoreType.DMA,
            pltpu.SemaphoreType.DMA,
            pltpu.SemaphoreType.DMA,
            pltpu.SemaphoreType.DMA,
            pltpu.SemaphoreType.DMA,
        ),
    )
    unsort = pl.kernel(
        _unsort_body,
        compiler_params=sc_params,
        out_type=jax.ShapeDtypeStruct((NW, VPC, 16), jnp.float32),
        mesh=mesh,
        scratch_types=(
            pltpu.VMEM((NP,), jnp.float32),
            pltpu.VMEM((VPC, 16), jnp.int32),
            pltpu.VMEM((VPC, 16), jnp.float32),
        ),
    )
    return route, unsort


def _pad_params(params):
    w1 = jnp.stack([jnp.pad(params[c][0][0], ((0, 0), (0, HP - H)))
                    for c in range(NCLS)])
    b1 = jnp.stack([jnp.pad(params[c][0][1], (0, HP - H))[None]
                    for c in range(NCLS)])
    w2 = jnp.stack([jnp.pad(params[c][1][0], ((0, HP - H), (0, HP - H)))
                    for c in range(NCLS)])
    b2 = jnp.stack([jnp.pad(params[c][1][1], (0, HP - H))[None]
                    for c in range(NCLS)])
    w3 = jnp.stack([jnp.pad(params[c][2][0], ((0, HP - H), (0, HP - H)))
                    for c in range(NCLS)])
    b3 = jnp.stack([jnp.pad(params[c][2][1], (0, HP - H))[None]
                    for c in range(NCLS)])
    w4 = jnp.stack([jnp.pad(params[c][3][0], ((0, HP - H), (0, OUTP - 1)))
                    for c in range(NCLS)])
    b4 = jnp.stack([jnp.pad(params[c][3][1], (0, OUTP - 1))[None]
                    for c in range(NCLS)])
    return w1, b1, w2, b2, w3, b3, w4, b4


def kernel(graph_emb, class_idx, params):
    w1, b1, w2, b2, w3, b3, w4, b4 = _pad_params(params)
    route, unsort = _sc_kernels()
    inv, tcls, xs = route(class_idx, graph_emb)
    res = _mlp_call(tcls[:NT], xs, w1, b1, w2, b2, w3, b3, w4, b4)
    out = unsort(res[:, 0], inv)
    return out.reshape(B)


# speedup vs baseline: 1.1423x; 1.0707x over previous
import functools

import jax
import jax.numpy as jnp
from jax import lax
from jax.experimental import pallas as pl
from jax.experimental.pallas import tpu as pltpu
from jax.experimental.pallas import tpu_sc as plsc

B = 16384
IN = 2048
NCLS = 3
H = 341
HP = 384
OUTP = 128
T = 256
TSHIFT = 8
NCORE = 2
NSUB = 16
NW = NCORE * NSUB
GR = 16

BH = B // 2
CHUNKH = BH // NW
VPCH = CHUNKH // 16
NPH = BH + 4 * T
NTH = NPH // T
NTPH = 48
IQ = CHUNKH // 128


def _make_route_body(off):
    def body(cls_hbm, emb_hbm, inv_hbm, tcls_hbm, xs_hbm,
             cls_v, pos_v, xa_v, xb_v, xc_v, tcls_v,
             sin0, sin1, sin2, sout0, sout1, sout2):
        wid = lax.axis_index("s") * NCORE + lax.axis_index("c")
        base = wid * CHUNKH
        pltpu.sync_copy(cls_hbm, cls_v)

        vzero = jnp.zeros((16,), jnp.int32)
        t0 = wid * VPCH

        def count_body(i, carry):
            a0, a1 = carry
            v = cls_v[pl.ds(i * 16, 16)]
            return (a0 + (v == 0).astype(jnp.int32),
                    a1 + (v == 1).astype(jnp.int32))

        acc0, acc1 = lax.fori_loop(0, t0, count_body, (vzero, vzero))
        pre0 = jnp.sum(acc0)
        pre1 = jnp.sum(acc1)
        acc0, acc1 = lax.fori_loop(t0, BH // 16, count_body, (acc0, acc1))
        tot0 = jnp.sum(acc0)
        tot1 = jnp.sum(acc1)
        pre2 = t0 * 16 - pre0 - pre1

        cap0 = ((tot0 + (T - 1)) >> TSHIFT) << TSHIFT
        cap1 = ((tot1 + (T - 1)) >> TSHIFT) << TSHIFT
        o1 = cap0
        o2 = cap0 + cap1

        @pl.when(wid == 0)
        def _emit_tile_classes():
            for k in range(NTPH // 16):
                tv = (lax.iota(jnp.int32, 16) + 16 * k) * T
                tcls_v[pl.ds(16 * k, 16)] = (
                    (tv >= o1).astype(jnp.int32) + (tv >= o2).astype(jnp.int32))
            pltpu.sync_copy(tcls_v, tcls_hbm)

        def pos_body(g, carry):
            b0, b1, b2 = carry
            v = cls_v[pl.ds((t0 + g) * 16, 16)]
            m0 = v == 0
            m1 = v == 1
            r0 = jnp.cumsum(m0.astype(jnp.int32))
            r1 = jnp.cumsum(m1.astype(jnp.int32))
            r2 = jnp.cumsum((1 - m0.astype(jnp.int32) - m1.astype(jnp.int32)))
            pos = jnp.where(m0, b0 + r0, jnp.where(m1, b1 + r1, b2 + r2)) - 1
            pos_v[g] = pos
            return (b0 + r0[15], b1 + r1[15], b2 + r2[15])

        lax.fori_loop(0, VPCH, pos_body, (pre0, o1 + pre1, o2 + pre2))
        pltpu.sync_copy(pos_v, inv_hbm.at[wid])

        bufs = (xa_v, xb_v, xc_v)
        sin = (sin0, sin1, sin2)
        sout = (sout0, sout1, sout2)

        def start_in(g, b):
            pltpu.async_copy(
                emb_hbm.at[pl.ds(off + base + g * GR, GR)], bufs[b], sin[b])

        def wait_in(g, b):
            pltpu.make_async_copy(
                emb_hbm.at[pl.ds(off + base + g * GR, GR)], bufs[b],
                sin[b]).wait()

        def start_out(g, b):
            pltpu.async_copy(bufs[b], xs_hbm.at[pos_v.at[g]], sout[b])

        def wait_out(g, b):
            pltpu.make_async_copy(
                bufs[b], xs_hbm.at[pos_v.at[g]], sout[b]).wait()

        NSTEP = VPCH
        FULL = (NSTEP - 1) // 3

        start_in(0, 0)

        def ring_body(jj, _):
            for b in range(3):
                g = 3 * jj + b
                if b == 2:
                    wait_out(g - 2, 0)
                else:
                    @pl.when(jj >= 1)
                    def _():
                        wait_out(g - 2, b + 1)
                start_in(g + 1, (b + 1) % 3)
                wait_in(g, b)
                start_out(g, b)
            return 0

        lax.fori_loop(0, FULL, ring_body, 0)
        for g in range(3 * FULL, NSTEP):
            if g >= 2:
                wait_out(g - 2, (g + 1) % 3)
            if g + 1 < NSTEP:
                start_in(g + 1, (g + 1) % 3)
            wait_in(g, g % 3)
            start_out(g, g % 3)
        for g in range(NSTEP - 2, NSTEP):
            wait_out(g, g % 3)

    return body


def _unsort_body(res_hbm, inv_hbm, out_hbm, inv_v, rows_v, out_v, sem):
    wid = lax.axis_index("s") * NCORE + lax.axis_index("c")
    pltpu.sync_copy(inv_hbm.at[wid], inv_v)
    for q in range(IQ):
        pltpu.async_copy(res_hbm.at[inv_v.at[q]],
                         rows_v.at[pl.ds(q * 128, 128)], sem)
    for q in range(IQ):
        pltpu.make_async_copy(res_hbm.at[inv_v.at[q]],
                              rows_v.at[pl.ds(q * 128, 128)], sem).wait()

    zeros16 = jnp.zeros((16,), jnp.int32)

    def body(g, _):
        rid = lax.iota(jnp.int32, 16) + g * 16
        out_v[g] = plsc.load_gather(rows_v, [rid, zeros16])
        return 0

    lax.fori_loop(0, VPCH, body, 0)
    pltpu.sync_copy(out_v, out_hbm.at[wid])


def _mlp_body(tcls_ref, x_ref, w1_ref, b1_ref, w2_ref, b2_ref,
              w3_ref, b3_ref, w4_ref, b4_ref, o_ref):
    x = x_ref[...].astype(jnp.bfloat16)
    h = jnp.dot(x, w1_ref[0], preferred_element_type=jnp.float32) + b1_ref[0]
    h = jnp.maximum(h, 0.0)
    h = jnp.dot(h, w2_ref[0], preferred_element_type=jnp.float32) + b2_ref[0]
    h = jnp.maximum(h, 0.0)
    h = jnp.dot(h, w3_ref[0], preferred_element_type=jnp.float32) + b3_ref[0]
    h = jnp.maximum(h, 0.0)
    o_ref[...] = (jnp.dot(h, w4_ref[0], preferred_element_type=jnp.float32)
                  + b4_ref[0])


_mlp_grid = pltpu.PrefetchScalarGridSpec(
    num_scalar_prefetch=1,
    grid=(NTH,),
    in_specs=[
        pl.BlockSpec((T, IN), lambda t, tc: (t, 0)),
        pl.BlockSpec((1, IN, HP), lambda t, tc: (tc[t], 0, 0)),
        pl.BlockSpec((1, 1, HP), lambda t, tc: (tc[t], 0, 0)),
        pl.BlockSpec((1, HP, HP), lambda t, tc: (tc[t], 0, 0)),
        pl.BlockSpec((1, 1, HP), lambda t, tc: (tc[t], 0, 0)),
        pl.BlockSpec((1, HP, HP), lambda t, tc: (tc[t], 0, 0)),
        pl.BlockSpec((1, 1, HP), lambda t, tc: (tc[t], 0, 0)),
        pl.BlockSpec((1, HP, OUTP), lambda t, tc: (tc[t], 0, 0)),
        pl.BlockSpec((1, 1, OUTP), lambda t, tc: (tc[t], 0, 0)),
    ],
    out_specs=pl.BlockSpec((T, OUTP), lambda t, tc: (t, 0)),
)

_mlp_call = pl.pallas_call(
    _mlp_body,
    grid_spec=_mlp_grid,
    out_shape=jax.ShapeDtypeStruct((NPH, OUTP), jnp.float32),
    compiler_params=pltpu.CompilerParams(
        dimension_semantics=("arbitrary",)),
)


@functools.lru_cache(maxsize=1)
def _sc_kernels():
    mesh = plsc.VectorSubcoreMesh(
        core_axis_name="c", subcore_axis_name="s",
        num_cores=NCORE, num_subcores=NSUB)
    sc_params = pltpu.CompilerParams(needs_layout_passes=False)
    route_out = (
        jax.ShapeDtypeStruct((NW, VPCH, 16), jnp.int32),
        jax.ShapeDtypeStruct((NTPH,), jnp.int32),
        jax.ShapeDtypeStruct((NPH, IN), jnp.float32),
    )
    route_scratch = (
        pltpu.VMEM((BH,), jnp.int32),
        pltpu.VMEM((VPCH, 16), jnp.int32),
        pltpu.VMEM((GR, IN), jnp.float32),
        pltpu.VMEM((GR, IN), jnp.float32),
        pltpu.VMEM((GR, IN), jnp.float32),
        pltpu.VMEM((NTPH,), jnp.int32),
        pltpu.SemaphoreType.DMA,
        pltpu.SemaphoreType.DMA,
        pltpu.SemaphoreType.DMA,
        pltpu.SemaphoreType.DMA,
        pltpu.SemaphoreType.DMA,
        pltpu.SemaphoreType.DMA,
    )
    routes = tuple(
        pl.kernel(
            _make_route_body(off),
            compiler_params=sc_params,
            out_type=route_out,
            mesh=mesh,
            scratch_types=route_scratch,
        )
        for off in (0, BH)
    )
    unsort = pl.kernel(
        _unsort_body,
        compiler_params=sc_params,
        out_type=jax.ShapeDtypeStruct((NW, VPCH, 16), jnp.float32),
        mesh=mesh,
        scratch_types=(
            pltpu.VMEM((IQ, 128), jnp.int32),
            pltpu.VMEM((CHUNKH, OUTP), jnp.float32),
            pltpu.VMEM((VPCH, 16), jnp.float32),
            pltpu.SemaphoreType.DMA,
        ),
    )
    return routes, unsort


def _pad_params(params):
    w1 = jnp.stack([jnp.pad(params[c][0][0], ((0, 0), (0, HP - H)))
                    for c in range(NCLS)]).astype(jnp.bfloat16)
    b1 = jnp.stack([jnp.pad(params[c][0][1], (0, HP - H))[None]
                    for c in range(NCLS)])
    w2 = jnp.stack([jnp.pad(params[c][1][0], ((0, HP - H), (0, HP - H)))
                    for c in range(NCLS)])
    b2 = jnp.stack([jnp.pad(params[c][1][1], (0, HP - H))[None]
                    for c in range(NCLS)])
    w3 = jnp.stack([jnp.pad(params[c][2][0], ((0, HP - H), (0, HP - H)))
                    for c in range(NCLS)])
    b3 = jnp.stack([jnp.pad(params[c][2][1], (0, HP - H))[None]
                    for c in range(NCLS)])
    w4 = jnp.stack([jnp.pad(params[c][3][0], ((0, HP - H), (0, OUTP - 1)))
                    for c in range(NCLS)])
    b4 = jnp.stack([jnp.pad(params[c][3][1], (0, OUTP - 1))[None]
                    for c in range(NCLS)])
    return w1, b1, w2, b2, w3, b3, w4, b4


def kernel(graph_emb, class_idx, params):
    ws = _pad_params(params)
    routes, unsort = _sc_kernels()
    halves = []
    for hi, route in enumerate(routes):
        cls_h = lax.slice_in_dim(class_idx, hi * BH, (hi + 1) * BH)
        halves.append(route(cls_h, graph_emb))
    outs = []
    for inv, tcls, xs in halves:
        res = _mlp_call(tcls[:NTH], xs, *ws)
        outs.append((res, inv))
    parts = []
    for res, inv in outs:
        o = unsort(res, inv.reshape(NW, IQ, 128))
        parts.append(o.reshape(BH))
    return jnp.concatenate(parts)


# speedup vs baseline: 1.1430x; 1.0006x over previous
import functools

import jax
import jax.numpy as jnp
from jax import lax
from jax.experimental import pallas as pl
from jax.experimental.pallas import tpu as pltpu
from jax.experimental.pallas import tpu_sc as plsc

B = 16384
IN = 2048
NCLS = 3
H = 341
HP = 384
OUTP = 128
T = 256
TSHIFT = 8
NCORE = 2
NSUB = 16
NW = NCORE * NSUB
GR = 16

BH = B // 2
CHUNKH = BH // NW
VPCH = CHUNKH // 16
NPH = BH + 4 * T
NTH = NPH // T
NTPH = 48
IQ = CHUNKH // 128


def _make_route_body(off):
    def body(cls_hbm, emb_hbm, inv_hbm, tcls_hbm, xs_hbm,
             cls_v, pos_v, xa_v, xb_v, xc_v, tcls_v,
             sin0, sin1, sin2, sout0, sout1, sout2):
        wid = lax.axis_index("s") * NCORE + lax.axis_index("c")
        base = wid * CHUNKH
        pltpu.sync_copy(cls_hbm, cls_v)

        vzero = jnp.zeros((16,), jnp.int32)
        t0 = wid * VPCH

        def count_body(i, carry):
            a0, a1 = carry
            v = cls_v[pl.ds(i * 16, 16)]
            return (a0 + (v == 0).astype(jnp.int32),
                    a1 + (v == 1).astype(jnp.int32))

        acc0, acc1 = lax.fori_loop(0, t0, count_body, (vzero, vzero))
        pre0 = jnp.sum(acc0)
        pre1 = jnp.sum(acc1)
        acc0, acc1 = lax.fori_loop(t0, BH // 16, count_body, (acc0, acc1))
        tot0 = jnp.sum(acc0)
        tot1 = jnp.sum(acc1)
        pre2 = t0 * 16 - pre0 - pre1

        cap0 = ((tot0 + (T - 1)) >> TSHIFT) << TSHIFT
        cap1 = ((tot1 + (T - 1)) >> TSHIFT) << TSHIFT
        o1 = cap0
        o2 = cap0 + cap1

        @pl.when(wid == 0)
        def _emit_tile_classes():
            for k in range(NTPH // 16):
                tv = (lax.iota(jnp.int32, 16) + 16 * k) * T
                tcls_v[pl.ds(16 * k, 16)] = (
                    (tv >= o1).astype(jnp.int32) + (tv >= o2).astype(jnp.int32))
            pltpu.sync_copy(tcls_v, tcls_hbm)

        def pos_body(g, carry):
            b0, b1, b2 = carry
            v = cls_v[pl.ds((t0 + g) * 16, 16)]
            m0 = v == 0
            m1 = v == 1
            r0 = jnp.cumsum(m0.astype(jnp.int32))
            r1 = jnp.cumsum(m1.astype(jnp.int32))
            r2 = jnp.cumsum((1 - m0.astype(jnp.int32) - m1.astype(jnp.int32)))
            pos = jnp.where(m0, b0 + r0, jnp.where(m1, b1 + r1, b2 + r2)) - 1
            pos_v[g] = pos
            return (b0 + r0[15], b1 + r1[15], b2 + r2[15])

        lax.fori_loop(0, VPCH, pos_body, (pre0, o1 + pre1, o2 + pre2))
        pltpu.sync_copy(pos_v, inv_hbm.at[wid])

        bufs = (xa_v, xb_v, xc_v)
        sin = (sin0, sin1, sin2)
        sout = (sout0, sout1, sout2)

        def start_in(g, b):
            pltpu.async_copy(
                emb_hbm.at[pl.ds(off + base + g * GR, GR)], bufs[b], sin[b])

        def wait_in(g, b):
            pltpu.make_async_copy(
                emb_hbm.at[pl.ds(off + base + g * GR, GR)], bufs[b],
                sin[b]).wait()

        def start_out(g, b):
            pltpu.async_copy(bufs[b], xs_hbm.at[pos_v.at[g]], sout[b])

        def wait_out(g, b):
            pltpu.make_async_copy(
                bufs[b], xs_hbm.at[pos_v.at[g]], sout[b]).wait()

        NSTEP = VPCH
        FULL = (NSTEP - 1) // 3

        start_in(0, 0)

        def ring_body(jj, _):
            for b in range(3):
                g = 3 * jj + b
                if b == 2:
                    wait_out(g - 2, 0)
                else:
                    @pl.when(jj >= 1)
                    def _():
                        wait_out(g - 2, b + 1)
                start_in(g + 1, (b + 1) % 3)
                wait_in(g, b)
                start_out(g, b)
            return 0

        lax.fori_loop(0, FULL, ring_body, 0)
        for g in range(3 * FULL, NSTEP):
            if g >= 2:
                wait_out(g - 2, (g + 1) % 3)
            if g + 1 < NSTEP:
                start_in(g + 1, (g + 1) % 3)
            wait_in(g, g % 3)
            start_out(g, g % 3)
        for g in range(NSTEP - 2, NSTEP):
            wait_out(g, g % 3)

    return body


def _unsort_body(res_hbm, inv_hbm, out_hbm, inv_v, rows_v, out_v, sem):
    wid = lax.axis_index("s") * NCORE + lax.axis_index("c")
    pltpu.sync_copy(inv_hbm.at[wid], inv_v)
    for q in range(IQ):
        pltpu.async_copy(res_hbm.at[inv_v.at[q]],
                         rows_v.at[pl.ds(q * 128, 128)], sem)
    for q in range(IQ):
        pltpu.make_async_copy(res_hbm.at[inv_v.at[q]],
                              rows_v.at[pl.ds(q * 128, 128)], sem).wait()

    zeros16 = jnp.zeros((16,), jnp.int32)

    def body(g, _):
        rid = lax.iota(jnp.int32, 16) + g * 16
        out_v[g] = plsc.load_gather(rows_v, [rid, zeros16])
        return 0

    lax.fori_loop(0, VPCH, body, 0)
    pltpu.sync_copy(out_v, out_hbm.at[wid])


def _mlp_body(tcls_ref, x_ref, w1_ref, b1_ref, w2_ref, b2_ref,
              w3_ref, b3_ref, w4_ref, b4_ref, o_ref):
    x = x_ref[...].astype(jnp.bfloat16)
    h = jnp.dot(x, w1_ref[0], preferred_element_type=jnp.float32) + b1_ref[0]
    h = jnp.maximum(h, 0.0)
    h = jnp.dot(h, w2_ref[0], preferred_element_type=jnp.float32) + b2_ref[0]
    h = jnp.maximum(h, 0.0)
    h = jnp.dot(h, w3_ref[0], preferred_element_type=jnp.float32) + b3_ref[0]
    h = jnp.maximum(h, 0.0)
    o_ref[...] = (jnp.dot(h, w4_ref[0], preferred_element_type=jnp.float32)
                  + b4_ref[0])


_mlp_grid = pltpu.PrefetchScalarGridSpec(
    num_scalar_prefetch=1,
    grid=(NTH,),
    in_specs=[
        pl.BlockSpec((T, IN), lambda t, tc: (t, 0)),
        pl.BlockSpec((1, IN, HP), lambda t, tc: (tc[t], 0, 0)),
        pl.BlockSpec((1, 1, HP), lambda t, tc: (tc[t], 0, 0)),
        pl.BlockSpec((1, HP, HP), lambda t, tc: (tc[t], 0, 0)),
        pl.BlockSpec((1, 1, HP), lambda t, tc: (tc[t], 0, 0)),
        pl.BlockSpec((1, HP, HP), lambda t, tc: (tc[t], 0, 0)),
        pl.BlockSpec((1, 1, HP), lambda t, tc: (tc[t], 0, 0)),
        pl.BlockSpec((1, HP, OUTP), lambda t, tc: (tc[t], 0, 0)),
        pl.BlockSpec((1, 1, OUTP), lambda t, tc: (tc[t], 0, 0)),
    ],
    out_specs=pl.BlockSpec((T, OUTP), lambda t, tc: (t, 0)),
)

_mlp_call = pl.pallas_call(
    _mlp_body,
    grid_spec=_mlp_grid,
    out_shape=jax.ShapeDtypeStruct((NPH, OUTP), jnp.float32),
    compiler_params=pltpu.CompilerParams(
        dimension_semantics=("arbitrary",)),
)


@functools.lru_cache(maxsize=1)
def _sc_kernels():
    mesh = plsc.VectorSubcoreMesh(
        core_axis_name="c", subcore_axis_name="s",
        num_cores=NCORE, num_subcores=NSUB)
    sc_params = pltpu.CompilerParams(
        needs_layout_passes=False, has_side_effects=True)
    route_out = (
        jax.ShapeDtypeStruct((NW, VPCH, 16), jnp.int32),
        jax.ShapeDtypeStruct((NTPH,), jnp.int32),
        jax.ShapeDtypeStruct((NPH, IN), jnp.float32),
    )
    route_scratch = (
        pltpu.VMEM((BH,), jnp.int32),
        pltpu.VMEM((VPCH, 16), jnp.int32),
        pltpu.VMEM((GR, IN), jnp.float32),
        pltpu.VMEM((GR, IN), jnp.float32),
        pltpu.VMEM((GR, IN), jnp.float32),
        pltpu.VMEM((NTPH,), jnp.int32),
        pltpu.SemaphoreType.DMA,
        pltpu.SemaphoreType.DMA,
        pltpu.SemaphoreType.DMA,
        pltpu.SemaphoreType.DMA,
        pltpu.SemaphoreType.DMA,
        pltpu.SemaphoreType.DMA,
    )
    routes = tuple(
        pl.kernel(
            _make_route_body(off),
            compiler_params=sc_params,
            out_type=route_out,
            mesh=mesh,
            scratch_types=route_scratch,
        )
        for off in (0, BH)
    )
    unsort = pl.kernel(
        _unsort_body,
        compiler_params=sc_params,
        out_type=jax.ShapeDtypeStruct((NW, VPCH, 16), jnp.float32),
        mesh=mesh,
        scratch_types=(
            pltpu.VMEM((IQ, 128), jnp.int32),
            pltpu.VMEM((CHUNKH, OUTP), jnp.float32),
            pltpu.VMEM((VPCH, 16), jnp.float32),
            pltpu.SemaphoreType.DMA,
        ),
    )
    return routes, unsort


def _pad_params(params):
    w1 = jnp.stack([jnp.pad(params[c][0][0], ((0, 0), (0, HP - H)))
                    for c in range(NCLS)]).astype(jnp.bfloat16)
    b1 = jnp.stack([jnp.pad(params[c][0][1], (0, HP - H))[None]
                    for c in range(NCLS)])
    w2 = jnp.stack([jnp.pad(params[c][1][0], ((0, HP - H), (0, HP - H)))
                    for c in range(NCLS)])
    b2 = jnp.stack([jnp.pad(params[c][1][1], (0, HP - H))[None]
                    for c in range(NCLS)])
    w3 = jnp.stack([jnp.pad(params[c][2][0], ((0, HP - H), (0, HP - H)))
                    for c in range(NCLS)])
    b3 = jnp.stack([jnp.pad(params[c][2][1], (0, HP - H))[None]
                    for c in range(NCLS)])
    w4 = jnp.stack([jnp.pad(params[c][3][0], ((0, HP - H), (0, OUTP - 1)))
                    for c in range(NCLS)])
    b4 = jnp.stack([jnp.pad(params[c][3][1], (0, OUTP - 1))[None]
                    for c in range(NCLS)])
    return w1, b1, w2, b2, w3, b3, w4, b4


def kernel(graph_emb, class_idx, params):
    ws = _pad_params(params)
    routes, unsort = _sc_kernels()
    halves = []
    for hi, route in enumerate(routes):
        cls_h = lax.slice_in_dim(class_idx, hi * BH, (hi + 1) * BH)
        halves.append(route(cls_h, graph_emb))
    outs = []
    for inv, tcls, xs in halves:
        res = _mlp_call(tcls[:NTH], xs, *ws)
        outs.append((res, inv))
    parts = []
    for res, inv in outs:
        o = unsort(res, inv.reshape(NW, IQ, 128))
        parts.append(o.reshape(BH))
    return jnp.concatenate(parts)


# speedup vs baseline: 1.1507x; 1.0068x over previous
import functools

import jax
import jax.numpy as jnp
from jax import lax
from jax.experimental import pallas as pl
from jax.experimental.pallas import tpu as pltpu
from jax.experimental.pallas import tpu_sc as plsc

B = 16384
IN = 2048
NCLS = 3
H = 341
HP = 384
OUTP = 128
T = 256
TSHIFT = 8
NCORE = 2
NSUB = 16
NW = NCORE * NSUB
GR = 16

BH = B // 2
CHUNKH = BH // NW
VPCH = CHUNKH // 16
NPH = BH + 4 * T
NTH = NPH // T
NTPH = 48
IQ = CHUNKH // 128


def _make_route_body(off):
    def body(cls_hbm, emb_hbm, inv_hbm, tcls_hbm, xs_hbm,
             cls_v, pos_v, pos2_v, xa_v, xb_v, xc_v, tcls_v,
             sin0, sin1, sin2, sout0, sout1, sout2):
        wid = lax.axis_index("s") * NCORE + lax.axis_index("c")
        base = wid * CHUNKH
        pltpu.sync_copy(cls_hbm, cls_v)

        vzero = jnp.zeros((16,), jnp.int32)
        t0 = wid * VPCH

        def count_body(i, carry):
            a0, a1 = carry
            v = cls_v[pl.ds(i * 16, 16)]
            return (a0 + (v == 0).astype(jnp.int32),
                    a1 + (v == 1).astype(jnp.int32))

        acc0, acc1 = lax.fori_loop(0, t0, count_body, (vzero, vzero))
        pre0 = jnp.sum(acc0)
        pre1 = jnp.sum(acc1)
        acc0, acc1 = lax.fori_loop(t0, BH // 16, count_body, (acc0, acc1))
        tot0 = jnp.sum(acc0)
        tot1 = jnp.sum(acc1)
        pre2 = t0 * 16 - pre0 - pre1

        cap0 = ((tot0 + (T - 1)) >> TSHIFT) << TSHIFT
        cap1 = ((tot1 + (T - 1)) >> TSHIFT) << TSHIFT
        o1 = cap0
        o2 = cap0 + cap1

        @pl.when(wid == 0)
        def _emit_tile_classes():
            for k in range(NTPH // 16):
                tv = (lax.iota(jnp.int32, 16) + 16 * k) * T
                tcls_v[pl.ds(16 * k, 16)] = (
                    (tv >= o1).astype(jnp.int32) + (tv >= o2).astype(jnp.int32))
            pltpu.sync_copy(tcls_v, tcls_hbm)

        def pos_body(g, carry):
            b0, b1, b2 = carry
            v = cls_v[pl.ds((t0 + g) * 16, 16)]
            m0 = v == 0
            m1 = v == 1
            r0 = jnp.cumsum(m0.astype(jnp.int32))
            r1 = jnp.cumsum(m1.astype(jnp.int32))
            r2 = jnp.cumsum((1 - m0.astype(jnp.int32) - m1.astype(jnp.int32)))
            pos = jnp.where(m0, b0 + r0, jnp.where(m1, b1 + r1, b2 + r2)) - 1
            pos_v[g] = pos
            return (b0 + r0[15], b1 + r1[15], b2 + r2[15])

        lax.fori_loop(0, VPCH, pos_body, (pre0, o1 + pre1, o2 + pre2))
        for g in range(VPCH):
            pos2_v[g // 8, pl.ds((g % 8) * 16, 16)] = pos_v[g]
        pltpu.sync_copy(pos2_v, inv_hbm.at[wid])

        bufs = (xa_v, xb_v, xc_v)
        sin = (sin0, sin1, sin2)
        sout = (sout0, sout1, sout2)

        def start_in(g, b):
            pltpu.async_copy(
                emb_hbm.at[pl.ds(off + base + g * GR, GR)], bufs[b], sin[b])

        def wait_in(g, b):
            pltpu.make_async_copy(
                emb_hbm.at[pl.ds(off + base + g * GR, GR)], bufs[b],
                sin[b]).wait()

        def start_out(g, b):
            pltpu.async_copy(bufs[b], xs_hbm.at[pos_v.at[g]], sout[b])

        def wait_out(g, b):
            pltpu.make_async_copy(
                bufs[b], xs_hbm.at[pos_v.at[g]], sout[b]).wait()

        NSTEP = VPCH
        FULL = (NSTEP - 1) // 3

        start_in(0, 0)

        def ring_body(jj, _):
            for b in range(3):
                g = 3 * jj + b
                if b == 2:
                    wait_out(g - 2, 0)
                else:
                    @pl.when(jj >= 1)
                    def _():
                        wait_out(g - 2, b + 1)
                start_in(g + 1, (b + 1) % 3)
                wait_in(g, b)
                start_out(g, b)
            return 0

        lax.fori_loop(0, FULL, ring_body, 0)
        for g in range(3 * FULL, NSTEP):
            if g >= 2:
                wait_out(g - 2, (g + 1) % 3)
            if g + 1 < NSTEP:
                start_in(g + 1, (g + 1) % 3)
            wait_in(g, g % 3)
            start_out(g, g % 3)
        for g in range(NSTEP - 2, NSTEP):
            wait_out(g, g % 3)

    return body


def _unsort_body(res0_hbm, res1_hbm, inv0_hbm, inv1_hbm, out_hbm,
                 inv_v, rows_v, out_v, sem):
    wid = lax.axis_index("s") * NCORE + lax.axis_index("c")
    zeros16 = jnp.zeros((16,), jnp.int32)
    for h, (res_hbm, inv_hbm) in enumerate(
            ((res0_hbm, inv0_hbm), (res1_hbm, inv1_hbm))):
        pltpu.sync_copy(inv_hbm.at[wid], inv_v)
        for q in range(IQ):
            pltpu.async_copy(res_hbm.at[inv_v.at[q]],
                             rows_v.at[pl.ds(q * 128, 128)], sem)
        for q in range(IQ):
            pltpu.make_async_copy(res_hbm.at[inv_v.at[q]],
                                  rows_v.at[pl.ds(q * 128, 128)], sem).wait()

        def body(g, _):
            rid = lax.iota(jnp.int32, 16) + g * 16
            out_v[g] = plsc.load_gather(rows_v, [rid, zeros16])
            return 0

        lax.fori_loop(0, VPCH, body, 0)
        pltpu.sync_copy(out_v, out_hbm.at[h, wid])


def _mlp_body(tcls_ref, x_ref, w1_ref, b1_ref, w2_ref, b2_ref,
              w3_ref, b3_ref, w4_ref, b4_ref, o_ref):
    x = x_ref[...].astype(jnp.bfloat16)
    h = jnp.dot(x, w1_ref[0], preferred_element_type=jnp.float32) + b1_ref[0]
    h = jnp.maximum(h, 0.0)
    h = jnp.dot(h, w2_ref[0], preferred_element_type=jnp.float32) + b2_ref[0]
    h = jnp.maximum(h, 0.0)
    h = jnp.dot(h, w3_ref[0], preferred_element_type=jnp.float32) + b3_ref[0]
    h = jnp.maximum(h, 0.0)
    o_ref[...] = (jnp.dot(h, w4_ref[0], preferred_element_type=jnp.float32)
                  + b4_ref[0])


_mlp_grid = pltpu.PrefetchScalarGridSpec(
    num_scalar_prefetch=1,
    grid=(NTH,),
    in_specs=[
        pl.BlockSpec((T, IN), lambda t, tc: (t, 0)),
        pl.BlockSpec((1, IN, HP), lambda t, tc: (tc[t], 0, 0)),
        pl.BlockSpec((1, 1, HP), lambda t, tc: (tc[t], 0, 0)),
        pl.BlockSpec((1, HP, HP), lambda t, tc: (tc[t], 0, 0)),
        pl.BlockSpec((1, 1, HP), lambda t, tc: (tc[t], 0, 0)),
        pl.BlockSpec((1, HP, HP), lambda t, tc: (tc[t], 0, 0)),
        pl.BlockSpec((1, 1, HP), lambda t, tc: (tc[t], 0, 0)),
        pl.BlockSpec((1, HP, OUTP), lambda t, tc: (tc[t], 0, 0)),
        pl.BlockSpec((1, 1, OUTP), lambda t, tc: (tc[t], 0, 0)),
    ],
    out_specs=pl.BlockSpec((T, OUTP), lambda t, tc: (t, 0)),
)

_mlp_call = pl.pallas_call(
    _mlp_body,
    grid_spec=_mlp_grid,
    out_shape=jax.ShapeDtypeStruct((NPH, OUTP), jnp.float32),
    compiler_params=pltpu.CompilerParams(
        dimension_semantics=("arbitrary",)),
)


@functools.lru_cache(maxsize=1)
def _sc_kernels():
    mesh = plsc.VectorSubcoreMesh(
        core_axis_name="c", subcore_axis_name="s",
        num_cores=NCORE, num_subcores=NSUB)
    sc_params = pltpu.CompilerParams(
        needs_layout_passes=False, has_side_effects=True)
    route_out = (
        jax.ShapeDtypeStruct((NW, IQ, 128), jnp.int32),
        jax.ShapeDtypeStruct((NTPH,), jnp.int32),
        jax.ShapeDtypeStruct((NPH, IN), jnp.float32),
    )
    route_scratch = (
        pltpu.VMEM((BH,), jnp.int32),
        pltpu.VMEM((VPCH, 16), jnp.int32),
        pltpu.VMEM((IQ, 128), jnp.int32),
        pltpu.VMEM((GR, IN), jnp.float32),
        pltpu.VMEM((GR, IN), jnp.float32),
        pltpu.VMEM((GR, IN), jnp.float32),
        pltpu.VMEM((NTPH,), jnp.int32),
        pltpu.SemaphoreType.DMA,
        pltpu.SemaphoreType.DMA,
        pltpu.SemaphoreType.DMA,
        pltpu.SemaphoreType.DMA,
        pltpu.SemaphoreType.DMA,
        pltpu.SemaphoreType.DMA,
    )
    routes = tuple(
        pl.kernel(
            _make_route_body(off),
            compiler_params=sc_params,
            out_type=route_out,
            mesh=mesh,
            scratch_types=route_scratch,
        )
        for off in (0, BH)
    )
    unsort = pl.kernel(
        _unsort_body,
        compiler_params=sc_params,
        out_type=jax.ShapeDtypeStruct((2, NW, VPCH, 16), jnp.float32),
        mesh=mesh,
        scratch_types=(
            pltpu.VMEM((IQ, 128), jnp.int32),
            pltpu.VMEM((CHUNKH, OUTP), jnp.float32),
            pltpu.VMEM((VPCH, 16), jnp.float32),
            pltpu.SemaphoreType.DMA,
        ),
    )
    return routes, unsort


def _pad_params(params):
    w1 = jnp.stack([jnp.pad(params[c][0][0], ((0, 0), (0, HP - H)))
                    for c in range(NCLS)]).astype(jnp.bfloat16)
    b1 = jnp.stack([jnp.pad(params[c][0][1], (0, HP - H))[None]
                    for c in range(NCLS)])
    w2 = jnp.stack([jnp.pad(params[c][1][0], ((0, HP - H), (0, HP - H)))
                    for c in range(NCLS)])
    b2 = jnp.stack([jnp.pad(params[c][1][1], (0, HP - H))[None]
                    for c in range(NCLS)])
    w3 = jnp.stack([jnp.pad(params[c][2][0], ((0, HP - H), (0, HP - H)))
                    for c in range(NCLS)])
    b3 = jnp.stack([jnp.pad(params[c][2][1], (0, HP - H))[None]
                    for c in range(NCLS)])
    w4 = jnp.stack([jnp.pad(params[c][3][0], ((0, HP - H), (0, OUTP - 1)))
                    for c in range(NCLS)])
    b4 = jnp.stack([jnp.pad(params[c][3][1], (0, OUTP - 1))[None]
                    for c in range(NCLS)])
    return w1, b1, w2, b2, w3, b3, w4, b4


def kernel(graph_emb, class_idx, params):
    ws = _pad_params(params)
    routes, unsort = _sc_kernels()
    halves = []
    for hi, route in enumerate(routes):
        cls_h = lax.slice_in_dim(class_idx, hi * BH, (hi + 1) * BH)
        halves.append(route(cls_h, graph_emb))
    res = [_mlp_call(tcls, xs, *ws) for _, tcls, xs in halves]
    out = unsort(res[0], res[1], halves[0][0], halves[1][0])
    return out.reshape(B)
